# baseline (speedup 1.0000x reference)
import functools

import jax
import jax.numpy as jnp
from jax import lax
from jax.experimental import pallas as pl
from jax.experimental.pallas import tpu as pltpu
from jax.experimental.pallas import tpu_sc as plsc

SEQ_OUT = 49
BATCH = 1024
VOCAB = 1000
EMB_ROWS = 100
EMB_DIM = 10

NUM_SC = 2
NUM_SUBCORES = 16
NUM_WORKERS = NUM_SC * NUM_SUBCORES
ROWS_PER_W = BATCH // NUM_WORKERS


def _tile_kernel(idx_ref, emb_ref, w_ref, b_ref, out_ref):
    idx = idx_ref[0]
    rows = jax.lax.broadcasted_iota(jnp.int32, (EMB_ROWS, BATCH), 0)
    onehot = (rows == idx).astype(jnp.float32)
    pooled = jax.lax.dot_general(
        onehot, emb_ref[:, :],
        dimension_numbers=(((0,), (0,)), ((), ())),
        preferred_element_type=jnp.float32,
    )
    out = jax.lax.dot_general(
        pooled, w_ref[:, :],
        dimension_numbers=(((1,), (0,)), ((), ())),
        preferred_element_type=jnp.float32,
    )
    out_ref[:, :] = out + b_ref[:, :]


def _compute_tile(answer, emb_table, lin_w, lin_b):
    idx = answer[:1].reshape(1, 1, BATCH).astype(jnp.int32)
    w_t = lin_w.T
    b2 = lin_b.reshape(1, VOCAB)
    return pl.pallas_call(
        _tile_kernel,
        out_shape=jax.ShapeDtypeStruct((BATCH, VOCAB), jnp.float32),
    )(idx, emb_table, w_t, b2)


def _sc_bcast_body(tile_hbm, out_hbm, rows_v, sem):
    wid = lax.axis_index("s") * NUM_SC + lax.axis_index("c")
    base = wid * ROWS_PER_W
    pltpu.sync_copy(tile_hbm.at[pl.ds(base, ROWS_PER_W), :], rows_v)

    def body(s, carry):
        pltpu.sync_copy(rows_v, out_hbm.at[s, pl.ds(base, ROWS_PER_W), :])
        return carry

    lax.fori_loop(0, SEQ_OUT, body, 0)


def kernel(question, answer, emb_table, lin_w, lin_b):
    del question
    tile = _compute_tile(answer, emb_table, lin_w, lin_b)

    sc_bcast = pl.kernel(
        _sc_bcast_body,
        out_type=jax.ShapeDtypeStruct((SEQ_OUT, BATCH, VOCAB), jnp.float32),
        mesh=plsc.VectorSubcoreMesh(core_axis_name="c", subcore_axis_name="s"),
        scratch_types=[
            pltpu.VMEM((ROWS_PER_W, VOCAB), jnp.float32),
            pltpu.SemaphoreType.DMA,
        ],
    )
    return sc_bcast(tile)
